# baseline (device time: 44199 ns/iter reference)
import jax
import jax.numpy as jnp
from jax import lax
from jax.experimental import pallas as pl
from jax.experimental.pallas import tpu as pltpu

N_DEV = 16
B = 256
D = 256
BLK = B // N_DEV


def kernel(x, Win0, Wout0, Win1, Wout1, Win2, Wout2):
    def body(
        x_ref,
        win0_ref,
        wout0_ref,
        win1_ref,
        wout1_ref,
        win2_ref,
        wout2_ref,
        out_ref,
        p_ref,
        r_ref,
        rs_buf,
        ag_buf,
        rs_sems,
        ag_sems,
        send_sems,
    ):
        my_id = lax.axis_index("i")

        def send_desc(j):
            return pltpu.make_async_remote_copy(
                src_ref=p_ref.at[pl.ds(j * BLK, BLK), :],
                dst_ref=rs_buf.at[my_id],
                send_sem=send_sems.at[j],
                recv_sem=rs_sems.at[my_id],
                device_id=(j,),
                device_id_type=pl.DeviceIdType.MESH,
            )

        def rs_recv_desc(s):
            return pltpu.make_async_remote_copy(
                src_ref=p_ref.at[pl.ds(0, BLK), :],
                dst_ref=rs_buf.at[s],
                send_sem=send_sems.at[s],
                recv_sem=rs_sems.at[s],
                device_id=(s,),
                device_id_type=pl.DeviceIdType.MESH,
            )

        def ag_send_desc(j):
            return pltpu.make_async_remote_copy(
                src_ref=r_ref,
                dst_ref=ag_buf.at[my_id],
                send_sem=send_sems.at[j],
                recv_sem=ag_sems.at[my_id],
                device_id=(j,),
                device_id_type=pl.DeviceIdType.MESH,
            )

        def ag_recv_desc(s):
            return pltpu.make_async_remote_copy(
                src_ref=r_ref,
                dst_ref=ag_buf.at[s],
                send_sem=send_sems.at[s],
                recv_sem=ag_sems.at[s],
                device_id=(s,),
                device_id_type=pl.DeviceIdType.MESH,
            )

        def reduce_scatter(P):
            p_ref[:, :] = P
            for j in range(N_DEV):

                @pl.when(my_id != j)
                def _(j=j):
                    send_desc(j).start()

                @pl.when(my_id == j)
                def _(j=j):
                    rs_buf[j, :, :] = p_ref[pl.ds(j * BLK, BLK), :]

            for s in range(N_DEV):

                @pl.when(my_id != s)
                def _(s=s):
                    rs_recv_desc(s).wait_recv()

            for j in range(N_DEV):

                @pl.when(my_id != j)
                def _(j=j):
                    send_desc(j).wait_send()

            return jnp.sum(rs_buf[:, :, :], axis=0)

        def all_gather(R):
            r_ref[:, :] = R
            for j in range(N_DEV):

                @pl.when(my_id != j)
                def _(j=j):
                    ag_send_desc(j).start()

                @pl.when(my_id == j)
                def _(j=j):
                    ag_buf[j, :, :] = r_ref[:, :]

            for s in range(N_DEV):

                @pl.when(my_id != s)
                def _(s=s):
                    ag_recv_desc(s).wait_recv()

            for j in range(N_DEV):

                @pl.when(my_id != j)
                def _(j=j):
                    ag_send_desc(j).wait_send()

            return ag_buf[:, :, :].reshape(B, D)

        x_val = x_ref[:, :]
        layers = [
            (win0_ref, wout0_ref),
            (win1_ref, wout1_ref),
            (win2_ref, wout2_ref),
        ]
        for l, (win, wout) in enumerate(layers):
            h = jnp.maximum(
                jnp.dot(x_val, win[:, :], preferred_element_type=jnp.float32),
                0.0,
            )
            P = jnp.dot(h, wout[:, :], preferred_element_type=jnp.float32)
            R = reduce_scatter(P)
            if l < len(layers) - 1:
                x_val = all_gather(R)
            else:
                out_ref[:, :] = R

    return pl.pallas_call(
        body,
        out_shape=jax.ShapeDtypeStruct((BLK, D), jnp.float32),
        in_specs=[pl.BlockSpec(memory_space=pltpu.VMEM)] * 7,
        out_specs=pl.BlockSpec(memory_space=pltpu.VMEM),
        scratch_shapes=[
            pltpu.VMEM((B, D), jnp.float32),
            pltpu.VMEM((BLK, D), jnp.float32),
            pltpu.VMEM((N_DEV, BLK, D), jnp.float32),
            pltpu.VMEM((N_DEV, BLK, D), jnp.float32),
            pltpu.SemaphoreType.DMA((N_DEV,)),
            pltpu.SemaphoreType.DMA((N_DEV,)),
            pltpu.SemaphoreType.DMA((N_DEV,)),
        ],
    )(x, Win0, Wout0, Win1, Wout1, Win2, Wout2)


# device time: 42625 ns/iter; 1.0369x vs baseline; 1.0369x over previous
import jax
import jax.numpy as jnp
from jax import lax
from jax.experimental import pallas as pl
from jax.experimental.pallas import tpu as pltpu

N_DEV = 16
B = 256
D = 256
BLK = B // N_DEV
G = 4
NG = N_DEV // G


def kernel(x, Win0, Wout0, Win1, Wout1, Win2, Wout2):
    def body(
        x_ref,
        win0_ref,
        wout0_ref,
        win1_ref,
        wout1_ref,
        win2_ref,
        wout2_ref,
        out_ref,
        p_ref,
        r_ref,
        rs_buf,
        ag_buf,
        rs_sems,
        ag_sems,
        rs_send_sems,
        ag_send_sems,
    ):
        my_id = lax.axis_index("i")

        def rs_send_desc(j):
            return pltpu.make_async_remote_copy(
                src_ref=p_ref.at[pl.ds(j * BLK, BLK), :],
                dst_ref=rs_buf.at[my_id],
                send_sem=rs_send_sems.at[j],
                recv_sem=rs_sems.at[my_id],
                device_id=(j,),
                device_id_type=pl.DeviceIdType.MESH,
            )

        def rs_recv_desc(s):
            return pltpu.make_async_remote_copy(
                src_ref=p_ref.at[pl.ds(0, BLK), :],
                dst_ref=rs_buf.at[s],
                send_sem=rs_send_sems.at[s],
                recv_sem=rs_sems.at[s],
                device_id=(s,),
                device_id_type=pl.DeviceIdType.MESH,
            )

        def ag_send_desc(j):
            return pltpu.make_async_remote_copy(
                src_ref=r_ref,
                dst_ref=ag_buf.at[my_id],
                send_sem=ag_send_sems.at[j],
                recv_sem=ag_sems.at[my_id],
                device_id=(j,),
                device_id_type=pl.DeviceIdType.MESH,
            )

        def ag_recv_desc(s):
            return pltpu.make_async_remote_copy(
                src_ref=r_ref,
                dst_ref=ag_buf.at[s],
                send_sem=ag_send_sems.at[s],
                recv_sem=ag_sems.at[s],
                device_id=(s,),
                device_id_type=pl.DeviceIdType.MESH,
            )

        def rs_start_block(s):

            @pl.when(my_id != s)
            def _():
                rs_send_desc(s).start()

            @pl.when(my_id == s)
            def _():
                rs_buf[s, :, :] = p_ref[pl.ds(s * BLK, BLK), :]

        def rs_finish():
            for s in range(N_DEV):

                @pl.when(my_id != s)
                def _(s=s):
                    rs_recv_desc(s).wait_recv()

            R = jnp.sum(rs_buf[:, :, :], axis=0)
            for j in range(N_DEV):

                @pl.when(my_id != j)
                def _(j=j):
                    rs_send_desc(j).wait_send()

            return R

        def ag_start(R, drain_prev):
            if drain_prev:
                for j in range(N_DEV):

                    @pl.when(my_id != j)
                    def _(j=j):
                        ag_send_desc(j).wait_send()

            r_ref[:, :] = R
            for j in range(N_DEV):

                @pl.when(my_id != j)
                def _(j=j):
                    ag_send_desc(j).start()

                @pl.when(my_id == j)
                def _(j=j):
                    ag_buf[j, :, :] = r_ref[:, :]

        def pipelined_layer(win_ref, wout_ref):
            win = win_ref[:, :]
            wout = wout_ref[:, :]
            for g in range(NG):
                for s in range(g * G, (g + 1) * G):

                    @pl.when(my_id != s)
                    def _(s=s):
                        ag_recv_desc(s).wait_recv()

                xg = ag_buf[pl.ds(g * G, G), :, :].reshape(G * BLK, D)
                h = jnp.maximum(
                    jnp.dot(xg, win, preferred_element_type=jnp.float32), 0.0
                )
                pg = jnp.dot(h, wout, preferred_element_type=jnp.float32)
                p_ref[pl.ds(g * G * BLK, G * BLK), :] = pg
                for s in range(g * G, (g + 1) * G):
                    rs_start_block(s)

        h0 = jnp.maximum(
            jnp.dot(x_ref[:, :], win0_ref[:, :], preferred_element_type=jnp.float32),
            0.0,
        )
        p_ref[:, :] = jnp.dot(
            h0, wout0_ref[:, :], preferred_element_type=jnp.float32
        )
        for s in range(N_DEV):
            rs_start_block(s)
        R = rs_finish()

        ag_start(R, drain_prev=False)
        pipelined_layer(win1_ref, wout1_ref)
        R = rs_finish()

        ag_start(R, drain_prev=True)
        pipelined_layer(win2_ref, wout2_ref)
        R = rs_finish()
        out_ref[:, :] = R

        for j in range(N_DEV):

            @pl.when(my_id != j)
            def _(j=j):
                ag_send_desc(j).wait_send()

    return pl.pallas_call(
        body,
        out_shape=jax.ShapeDtypeStruct((BLK, D), jnp.float32),
        in_specs=[pl.BlockSpec(memory_space=pltpu.VMEM)] * 7,
        out_specs=pl.BlockSpec(memory_space=pltpu.VMEM),
        scratch_shapes=[
            pltpu.VMEM((B, D), jnp.float32),
            pltpu.VMEM((BLK, D), jnp.float32),
            pltpu.VMEM((N_DEV, BLK, D), jnp.float32),
            pltpu.VMEM((N_DEV, BLK, D), jnp.float32),
            pltpu.SemaphoreType.DMA((N_DEV,)),
            pltpu.SemaphoreType.DMA((N_DEV,)),
            pltpu.SemaphoreType.DMA((N_DEV,)),
            pltpu.SemaphoreType.DMA((N_DEV,)),
        ],
    )(x, Win0, Wout0, Win1, Wout1, Win2, Wout2)
